# baseline (speedup 1.0000x reference)
import functools

import jax
import jax.numpy as jnp
from jax.experimental import pallas as pl
from jax.experimental.pallas import tpu as pltpu

N = 207
E = 2000
B = 4
F_IN = 2
H = 16
T = 12
NH = N * H
TB = T * B
NBLK = 27
RB = (3 * NH) // NBLK
EC = 16
EPAD = EC * 128


def _gcn_gi_kernel(wih_ref, x_ref, edge_ref, w1_ref, b1_ref, w2_ref, b2_ref,
                   w3_ref, b3_ref, beff_ref, whh_ref, gi_ref, whhbf_ref,
                   a_sc, seqT_sc):
    k = pl.program_id(0)

    @pl.when(k == 0)
    def _prologue():
        a_sc[...] = jnp.zeros((N, N), jnp.float32)

        def edge_chunk(c, carry):
            ch = edge_ref[:, pl.ds(c * 128, 128)]
            s_row = ch[0:1, :]
            d_row = ch[1:2, :]
            s_col = jnp.transpose(s_row)
            src_oh_t = jnp.where(
                jax.lax.broadcasted_iota(jnp.int32, (128, N), 1) == s_col,
                1.0, 0.0).astype(jnp.float32)
            dst_oh = jnp.where(
                jax.lax.broadcasted_iota(jnp.int32, (N, 128), 0) == d_row,
                1.0, 0.0).astype(jnp.float32)
            a_sc[...] += jnp.dot(dst_oh, src_oh_t,
                                 preferred_element_type=jnp.float32)
            return carry

        jax.lax.fori_loop(0, EC, edge_chunk, 0)

        a = a_sc[...]
        out_deg = jnp.sum(a, axis=0, keepdims=True)
        in_deg = jnp.sum(a, axis=1, keepdims=True)
        a_norm = (a * jax.lax.rsqrt(jnp.maximum(out_deg, 1.0))
                    * jax.lax.rsqrt(jnp.maximum(in_deg, 1.0)))
        a_sc[...] = a_norm

        t1 = jnp.dot(a_norm, x_ref[...], preferred_element_type=jnp.float32)
        h1 = jax.nn.relu(jnp.dot(t1, w1_ref[...],
                                 preferred_element_type=jnp.float32)
                         + b1_ref[...])
        t2 = jnp.dot(a_sc[...], h1, preferred_element_type=jnp.float32)
        h2 = jax.nn.relu(jnp.dot(t2, w2_ref[...],
                                 preferred_element_type=jnp.float32)
                         + b2_ref[...])
        t3 = jnp.dot(a_sc[...], h2, preferred_element_type=jnp.float32)
        h3 = (jnp.dot(t3, w3_ref[...], preferred_element_type=jnp.float32)
              + b3_ref[...])
        seqT_sc[...] = jnp.transpose(
            h3.reshape(N, TB, H), (0, 2, 1)).reshape(NH, TB)

    gi_blk = jnp.dot(wih_ref[...], seqT_sc[...],
                     preferred_element_type=jnp.float32)
    gi_ref[0, :, :] = (jnp.transpose(gi_blk)
                       + beff_ref[0, :, :]).astype(jnp.bfloat16)
    whhbf_ref[0, :, :] = whh_ref[0, :, :].astype(jnp.bfloat16)


def _gru_kernel(wT_hbm, gir_ref, giz_ref, gin_ref, bn_ref, out_ref,
                wT_sc, h_sc, sem):
    t = pl.program_id(0)
    bn = bn_ref[...]
    gi_r = gir_ref[0, :, :].astype(jnp.float32)
    gi_z = giz_ref[0, :, :].astype(jnp.float32)
    gi_n = gin_ref[0, :, :].astype(jnp.float32)

    @pl.when(t == 0)
    def _first():
        cp = pltpu.make_async_copy(wT_hbm, wT_sc, sem)
        cp.start()
        r = jax.nn.sigmoid(gi_r)
        z = jax.nn.sigmoid(gi_z)
        n = jnp.tanh(gi_n + r * bn)
        h_sc[...] = (1.0 - z) * n
        cp.wait()

    @pl.when(t > 0)
    def _step():
        h = h_sc[...]
        hb = h.astype(jnp.bfloat16)
        gh_r = jnp.dot(hb, wT_sc[0, :, :],
                       preferred_element_type=jnp.float32)
        gh_z = jnp.dot(hb, wT_sc[1, :, :],
                       preferred_element_type=jnp.float32)
        gh_n = jnp.dot(hb, wT_sc[2, :, :],
                       preferred_element_type=jnp.float32)
        r = jax.nn.sigmoid(gi_r + gh_r)
        z = jax.nn.sigmoid(gi_z + gh_z)
        n = jnp.tanh(gi_n + r * (gh_n + bn))
        h_sc[...] = (1.0 - z) * n + z * h

    @pl.when(t == T - 1)
    def _emit():
        out_ref[...] = h_sc[...]


@functools.partial(jax.jit, static_argnames=("interpret",))
def kernel(inputs, edge_index, W1, b1, W2, b2, W3, b3, W_ih, W_hh, b_ih, b_hh,
           interpret=False):
    f32 = jnp.float32
    x_r = jnp.transpose(inputs, (0, 3, 1, 2)).reshape(N, TB * F_IN)
    edge_p = jnp.pad(edge_index, ((0, 0), (0, EPAD - E)), constant_values=N)
    eye_tb = jnp.eye(TB, dtype=f32)
    w1k = jnp.kron(eye_tb, W1)
    w2k = jnp.kron(eye_tb, W2)
    w3k = jnp.kron(eye_tb, W3)
    b1k = jnp.tile(b1, TB)[None, :]
    b2k = jnp.tile(b2, TB)[None, :]
    b3k = jnp.tile(b3, TB)[None, :]
    b_eff = b_ih + jnp.concatenate([b_hh[:2 * NH], jnp.zeros(NH, f32)])
    b_eff_bc = jnp.broadcast_to(b_eff.reshape(NBLK, 1, RB), (NBLK, TB, RB))
    b_n_bc = jnp.broadcast_to(b_hh[2 * NH:][None, :], (B, NH))
    whh_r = W_hh.reshape(3, NH, NH)

    gi18, whh_bf = pl.pallas_call(
        _gcn_gi_kernel,
        grid=(NBLK,),
        in_specs=[
            pl.BlockSpec((RB, NH), lambda k: (k, 0)),
            pl.BlockSpec((N, TB * F_IN), lambda k: (0, 0)),
            pl.BlockSpec((2, EPAD), lambda k: (0, 0)),
            pl.BlockSpec((TB * F_IN, TB * H), lambda k: (0, 0)),
            pl.BlockSpec((1, TB * H), lambda k: (0, 0)),
            pl.BlockSpec((TB * H, TB * H), lambda k: (0, 0)),
            pl.BlockSpec((1, TB * H), lambda k: (0, 0)),
            pl.BlockSpec((TB * H, TB * H), lambda k: (0, 0)),
            pl.BlockSpec((1, TB * H), lambda k: (0, 0)),
            pl.BlockSpec((1, TB, RB), lambda k: (k, 0, 0)),
            pl.BlockSpec((1, RB, NH), lambda k: (k // 9, k % 9, 0)),
        ],
        out_specs=[
            pl.BlockSpec((1, TB, RB), lambda k: (k, 0, 0)),
            pl.BlockSpec((1, RB, NH), lambda k: (k // 9, k % 9, 0)),
        ],
        out_shape=[
            jax.ShapeDtypeStruct((NBLK, TB, RB), jnp.bfloat16),
            jax.ShapeDtypeStruct((3, NH, NH), jnp.bfloat16),
        ],
        scratch_shapes=[
            pltpu.VMEM((N, N), f32),
            pltpu.VMEM((NH, TB), f32),
        ],
        compiler_params=pltpu.CompilerParams(
            dimension_semantics=("arbitrary",),
            vmem_limit_bytes=128 * 1024 * 1024,
        ),
        interpret=interpret,
    )(W_ih, x_r, edge_p, w1k, b1k, w2k, b2k, w3k, b3k, b_eff_bc, whh_r)

    gi3 = jnp.transpose(gi18.reshape(3, 9, TB, RB),
                        (0, 2, 1, 3)).reshape(3, TB, NH)
    gi_t = gi3.reshape(3, T, B, NH)
    whhT = jnp.transpose(whh_bf, (0, 2, 1))

    h_final = pl.pallas_call(
        _gru_kernel,
        grid=(T,),
        in_specs=[
            pl.BlockSpec(memory_space=pl.ANY),
            pl.BlockSpec((1, B, NH), lambda t: (t, 0, 0)),
            pl.BlockSpec((1, B, NH), lambda t: (t, 0, 0)),
            pl.BlockSpec((1, B, NH), lambda t: (t, 0, 0)),
            pl.BlockSpec((B, NH), lambda t: (0, 0)),
        ],
        out_specs=pl.BlockSpec((B, NH), lambda t: (0, 0)),
        out_shape=jax.ShapeDtypeStruct((B, NH), f32),
        scratch_shapes=[
            pltpu.VMEM((3, NH, NH), jnp.bfloat16),
            pltpu.VMEM((B, NH), f32),
            pltpu.SemaphoreType.DMA,
        ],
        compiler_params=pltpu.CompilerParams(
            dimension_semantics=("arbitrary",),
            vmem_limit_bytes=128 * 1024 * 1024,
        ),
        interpret=interpret,
    )(whhT, gi_t[0], gi_t[1], gi_t[2], b_n_bc)

    return h_final.reshape(B, N, H)


# speedup vs baseline: 1.0939x; 1.0939x over previous
import functools

import jax
import jax.numpy as jnp
from jax.experimental import pallas as pl
from jax.experimental.pallas import tpu as pltpu

N = 207
E = 2000
B = 4
F_IN = 2
H = 16
T = 12
NH = N * H
TB = T * B
NBLK = 27
RB = (3 * NH) // NBLK
EC = 16
EPAD = EC * 128


def _gcn_gi_kernel(wih_ref, x_ref, edge_ref, w1_ref, b1_ref, w2_ref, b2_ref,
                   w3_ref, b3_ref, beff_ref, gi_ref, a_sc, seqT_sc):
    k = pl.program_id(0)

    @pl.when(k == 0)
    def _prologue():
        a_sc[...] = jnp.zeros((N, N), jnp.float32)

        def edge_chunk(c, carry):
            ch = edge_ref[:, pl.ds(c * 128, 128)]
            s_row = ch[0:1, :]
            d_row = ch[1:2, :]
            s_col = jnp.transpose(s_row)
            src_oh_t = jnp.where(
                jax.lax.broadcasted_iota(jnp.int32, (128, N), 1) == s_col,
                1.0, 0.0).astype(jnp.float32)
            dst_oh = jnp.where(
                jax.lax.broadcasted_iota(jnp.int32, (N, 128), 0) == d_row,
                1.0, 0.0).astype(jnp.float32)
            a_sc[...] += jnp.dot(dst_oh, src_oh_t,
                                 preferred_element_type=jnp.float32)
            return carry

        jax.lax.fori_loop(0, EC, edge_chunk, 0)

        a = a_sc[...]
        out_deg = jnp.sum(a, axis=0, keepdims=True)
        in_deg = jnp.sum(a, axis=1, keepdims=True)
        a_norm = (a * jax.lax.rsqrt(jnp.maximum(out_deg, 1.0))
                    * jax.lax.rsqrt(jnp.maximum(in_deg, 1.0)))
        a_sc[...] = a_norm

        t1 = jnp.dot(a_norm, x_ref[...], preferred_element_type=jnp.float32)
        h1 = jax.nn.relu(jnp.dot(t1, w1_ref[...],
                                 preferred_element_type=jnp.float32)
                         + b1_ref[...])
        t2 = jnp.dot(a_sc[...], h1, preferred_element_type=jnp.float32)
        h2 = jax.nn.relu(jnp.dot(t2, w2_ref[...],
                                 preferred_element_type=jnp.float32)
                         + b2_ref[...])
        t3 = jnp.dot(a_sc[...], h2, preferred_element_type=jnp.float32)
        h3 = (jnp.dot(t3, w3_ref[...], preferred_element_type=jnp.float32)
              + b3_ref[...])
        seqT_sc[...] = jnp.transpose(
            h3.reshape(N, TB, H), (0, 2, 1)).reshape(NH, TB)

    gi_blk = jnp.dot(wih_ref[...], seqT_sc[...],
                     preferred_element_type=jnp.float32)
    gi_ref[0, :, :] = (jnp.transpose(gi_blk)
                       + beff_ref[0, :, :]).astype(jnp.bfloat16)


def _gru_kernel(wT_hbm, gir_ref, giz_ref, gin_ref, bn_ref, out_ref,
                wT_sc, h_sc, sem):
    t = pl.program_id(0)
    bn = bn_ref[...]
    gi_r = gir_ref[0, :, :].astype(jnp.float32)
    gi_z = giz_ref[0, :, :].astype(jnp.float32)
    gi_n = gin_ref[0, :, :].astype(jnp.float32)

    @pl.when(t == 0)
    def _first():
        cp = pltpu.make_async_copy(wT_hbm, wT_sc, sem)
        cp.start()
        r = jax.nn.sigmoid(gi_r)
        z = jax.nn.sigmoid(gi_z)
        n = jnp.tanh(gi_n + r * bn)
        h_sc[...] = (1.0 - z) * n
        cp.wait()

    @pl.when(t > 0)
    def _step():
        h = h_sc[...]
        hb = h.astype(jnp.bfloat16)
        gh_r = jnp.dot(hb, wT_sc[0, :, :],
                       preferred_element_type=jnp.float32)
        gh_z = jnp.dot(hb, wT_sc[1, :, :],
                       preferred_element_type=jnp.float32)
        gh_n = jnp.dot(hb, wT_sc[2, :, :],
                       preferred_element_type=jnp.float32)
        r = jax.nn.sigmoid(gi_r + gh_r)
        z = jax.nn.sigmoid(gi_z + gh_z)
        n = jnp.tanh(gi_n + r * (gh_n + bn))
        h_sc[...] = (1.0 - z) * n + z * h

    @pl.when(t == T - 1)
    def _emit():
        out_ref[...] = h_sc[...]


@functools.partial(jax.jit, static_argnames=("interpret",))
def kernel(inputs, edge_index, W1, b1, W2, b2, W3, b3, W_ih, W_hh, b_ih, b_hh,
           interpret=False):
    f32 = jnp.float32
    x_r = jnp.transpose(inputs, (0, 3, 1, 2)).reshape(N, TB * F_IN)
    edge_p = jnp.pad(edge_index, ((0, 0), (0, EPAD - E)), constant_values=N)
    eye_tb = jnp.eye(TB, dtype=f32)
    w1k = jnp.kron(eye_tb, W1)
    w2k = jnp.kron(eye_tb, W2)
    w3k = jnp.kron(eye_tb, W3)
    b1k = jnp.tile(b1, TB)[None, :]
    b2k = jnp.tile(b2, TB)[None, :]
    b3k = jnp.tile(b3, TB)[None, :]
    b_eff = b_ih + jnp.concatenate([b_hh[:2 * NH], jnp.zeros(NH, f32)])
    b_eff_bc = jnp.broadcast_to(b_eff.reshape(NBLK, 1, RB), (NBLK, TB, RB))
    b_n_bc = jnp.broadcast_to(b_hh[2 * NH:][None, :], (B, NH))
    whhT = jnp.transpose(W_hh.reshape(3, NH, NH), (0, 2, 1)).astype(
        jnp.bfloat16)

    gi18 = pl.pallas_call(
        _gcn_gi_kernel,
        grid=(NBLK,),
        in_specs=[
            pl.BlockSpec((RB, NH), lambda k: (k, 0)),
            pl.BlockSpec((N, TB * F_IN), lambda k: (0, 0)),
            pl.BlockSpec((2, EPAD), lambda k: (0, 0)),
            pl.BlockSpec((TB * F_IN, TB * H), lambda k: (0, 0)),
            pl.BlockSpec((1, TB * H), lambda k: (0, 0)),
            pl.BlockSpec((TB * H, TB * H), lambda k: (0, 0)),
            pl.BlockSpec((1, TB * H), lambda k: (0, 0)),
            pl.BlockSpec((TB * H, TB * H), lambda k: (0, 0)),
            pl.BlockSpec((1, TB * H), lambda k: (0, 0)),
            pl.BlockSpec((1, TB, RB), lambda k: (k, 0, 0)),
        ],
        out_specs=pl.BlockSpec((1, TB, RB), lambda k: (k, 0, 0)),
        out_shape=jax.ShapeDtypeStruct((NBLK, TB, RB), jnp.bfloat16),
        scratch_shapes=[
            pltpu.VMEM((N, N), f32),
            pltpu.VMEM((NH, TB), f32),
        ],
        compiler_params=pltpu.CompilerParams(
            dimension_semantics=("arbitrary",),
            vmem_limit_bytes=128 * 1024 * 1024,
        ),
        interpret=interpret,
    )(W_ih, x_r, edge_p, w1k, b1k, w2k, b2k, w3k, b3k, b_eff_bc)

    gi3 = jnp.transpose(gi18.reshape(3, 9, TB, RB),
                        (0, 2, 1, 3)).reshape(3, TB, NH)
    gi_t = gi3.reshape(3, T, B, NH)

    h_final = pl.pallas_call(
        _gru_kernel,
        grid=(T,),
        in_specs=[
            pl.BlockSpec(memory_space=pl.ANY),
            pl.BlockSpec((1, B, NH), lambda t: (t, 0, 0)),
            pl.BlockSpec((1, B, NH), lambda t: (t, 0, 0)),
            pl.BlockSpec((1, B, NH), lambda t: (t, 0, 0)),
            pl.BlockSpec((B, NH), lambda t: (0, 0)),
        ],
        out_specs=pl.BlockSpec((B, NH), lambda t: (0, 0)),
        out_shape=jax.ShapeDtypeStruct((B, NH), f32),
        scratch_shapes=[
            pltpu.VMEM((3, NH, NH), jnp.bfloat16),
            pltpu.VMEM((B, NH), f32),
            pltpu.SemaphoreType.DMA,
        ],
        compiler_params=pltpu.CompilerParams(
            dimension_semantics=("arbitrary",),
            vmem_limit_bytes=128 * 1024 * 1024,
        ),
        interpret=interpret,
    )(whhT, gi_t[0], gi_t[1], gi_t[2], b_n_bc)

    return h_final.reshape(B, N, H)


# speedup vs baseline: 1.1121x; 1.0166x over previous
import functools

import jax
import jax.numpy as jnp
from jax.experimental import pallas as pl
from jax.experimental.pallas import tpu as pltpu

N = 207
E = 2000
B = 4
F_IN = 2
H = 16
T = 12
NH = N * H
TB = T * B
NBLK = 18
RB = (3 * NH) // NBLK
EC = 16
EPAD = EC * 128


def _gcn_gi_kernel(wih_ref, x_ref, edge_ref, w1_ref, b1_ref, w2_ref, b2_ref,
                   w3_ref, b3_ref, beff_ref, gi_ref, a_sc, seqT_sc):
    k = pl.program_id(0)

    @pl.when(k == 0)
    def _prologue():
        a_sc[...] = jnp.zeros((N, N), jnp.float32)

        def edge_chunk(c, carry):
            ch = edge_ref[:, pl.ds(c * 128, 128)]
            s_row = ch[0:1, :]
            d_row = ch[1:2, :]
            s_col = jnp.transpose(s_row)
            src_oh_t = jnp.where(
                jax.lax.broadcasted_iota(jnp.int32, (128, N), 1) == s_col,
                1.0, 0.0).astype(jnp.float32)
            dst_oh = jnp.where(
                jax.lax.broadcasted_iota(jnp.int32, (N, 128), 0) == d_row,
                1.0, 0.0).astype(jnp.float32)
            a_sc[...] += jnp.dot(dst_oh, src_oh_t,
                                 preferred_element_type=jnp.float32)
            return carry

        jax.lax.fori_loop(0, EC, edge_chunk, 0)

        a = a_sc[...]
        out_deg = jnp.sum(a, axis=0, keepdims=True)
        in_deg = jnp.sum(a, axis=1, keepdims=True)
        a_norm = (a * jax.lax.rsqrt(jnp.maximum(out_deg, 1.0))
                    * jax.lax.rsqrt(jnp.maximum(in_deg, 1.0)))
        a_sc[...] = a_norm

        t1 = jnp.dot(a_norm, x_ref[...], preferred_element_type=jnp.float32)
        h1 = jax.nn.relu(jnp.dot(t1, w1_ref[...],
                                 preferred_element_type=jnp.float32)
                         + b1_ref[...])
        t2 = jnp.dot(a_sc[...], h1, preferred_element_type=jnp.float32)
        h2 = jax.nn.relu(jnp.dot(t2, w2_ref[...],
                                 preferred_element_type=jnp.float32)
                         + b2_ref[...])
        t3 = jnp.dot(a_sc[...], h2, preferred_element_type=jnp.float32)
        h3 = (jnp.dot(t3, w3_ref[...], preferred_element_type=jnp.float32)
              + b3_ref[...])
        seqT_sc[...] = jnp.transpose(
            h3.reshape(N, TB, H), (0, 2, 1)).reshape(NH, TB)

    gi_blk = jnp.dot(wih_ref[...], seqT_sc[...],
                     preferred_element_type=jnp.float32)
    gi_ref[0, :, :] = (jnp.transpose(gi_blk)
                       + beff_ref[0, :, :]).astype(jnp.bfloat16)


def _gru_kernel(wT_hbm, gir_ref, giz_ref, gin_ref, bn_ref, out_ref,
                wT_sc, h_sc, sem):
    t = pl.program_id(0)
    bn = bn_ref[...]
    gi_r = gir_ref[0, :, :].astype(jnp.float32)
    gi_z = giz_ref[0, :, :].astype(jnp.float32)
    gi_n = gin_ref[0, :, :].astype(jnp.float32)

    @pl.when(t == 0)
    def _first():
        cp = pltpu.make_async_copy(wT_hbm, wT_sc, sem)
        cp.start()
        r = jax.nn.sigmoid(gi_r)
        z = jax.nn.sigmoid(gi_z)
        n = jnp.tanh(gi_n + r * bn)
        h_sc[...] = (1.0 - z) * n
        cp.wait()

    @pl.when(t > 0)
    def _step():
        h = h_sc[...]
        hb = h.astype(jnp.bfloat16)
        gh_r = jnp.dot(hb, wT_sc[0, :, :],
                       preferred_element_type=jnp.float32)
        gh_z = jnp.dot(hb, wT_sc[1, :, :],
                       preferred_element_type=jnp.float32)
        gh_n = jnp.dot(hb, wT_sc[2, :, :],
                       preferred_element_type=jnp.float32)
        r = jax.nn.sigmoid(gi_r + gh_r)
        z = jax.nn.sigmoid(gi_z + gh_z)
        n = jnp.tanh(gi_n + r * (gh_n + bn))
        h_sc[...] = (1.0 - z) * n + z * h

    @pl.when(t == T - 1)
    def _emit():
        out_ref[...] = h_sc[...]


@functools.partial(jax.jit, static_argnames=("interpret",))
def kernel(inputs, edge_index, W1, b1, W2, b2, W3, b3, W_ih, W_hh, b_ih, b_hh,
           interpret=False):
    f32 = jnp.float32
    x_r = jnp.transpose(inputs, (0, 3, 1, 2)).reshape(N, TB * F_IN)
    edge_p = jnp.pad(edge_index, ((0, 0), (0, EPAD - E)), constant_values=N)
    eye_tb = jnp.eye(TB, dtype=f32)
    w1k = jnp.kron(eye_tb, W1)
    w2k = jnp.kron(eye_tb, W2)
    w3k = jnp.kron(eye_tb, W3)
    b1k = jnp.tile(b1, TB)[None, :]
    b2k = jnp.tile(b2, TB)[None, :]
    b3k = jnp.tile(b3, TB)[None, :]
    b_eff = b_ih + jnp.concatenate([b_hh[:2 * NH], jnp.zeros(NH, f32)])
    b_eff_bc = jnp.broadcast_to(b_eff.reshape(NBLK, 1, RB), (NBLK, TB, RB))
    b_n_bc = jnp.broadcast_to(b_hh[2 * NH:][None, :], (B, NH))
    whhT = jnp.transpose(W_hh.reshape(3, NH, NH), (0, 2, 1)).astype(
        jnp.bfloat16)

    gi18 = pl.pallas_call(
        _gcn_gi_kernel,
        grid=(NBLK,),
        in_specs=[
            pl.BlockSpec((RB, NH), lambda k: (k, 0)),
            pl.BlockSpec((N, TB * F_IN), lambda k: (0, 0)),
            pl.BlockSpec((2, EPAD), lambda k: (0, 0)),
            pl.BlockSpec((TB * F_IN, TB * H), lambda k: (0, 0)),
            pl.BlockSpec((1, TB * H), lambda k: (0, 0)),
            pl.BlockSpec((TB * H, TB * H), lambda k: (0, 0)),
            pl.BlockSpec((1, TB * H), lambda k: (0, 0)),
            pl.BlockSpec((TB * H, TB * H), lambda k: (0, 0)),
            pl.BlockSpec((1, TB * H), lambda k: (0, 0)),
            pl.BlockSpec((1, TB, RB), lambda k: (k, 0, 0)),
        ],
        out_specs=pl.BlockSpec((1, TB, RB), lambda k: (k, 0, 0)),
        out_shape=jax.ShapeDtypeStruct((NBLK, TB, RB), jnp.bfloat16),
        scratch_shapes=[
            pltpu.VMEM((N, N), f32),
            pltpu.VMEM((NH, TB), f32),
        ],
        compiler_params=pltpu.CompilerParams(
            dimension_semantics=("arbitrary",),
            vmem_limit_bytes=128 * 1024 * 1024,
        ),
        interpret=interpret,
    )(W_ih, x_r, edge_p, w1k, b1k, w2k, b2k, w3k, b3k, b_eff_bc)

    gi3 = jnp.transpose(gi18.reshape(3, NBLK // 3, TB, RB),
                        (0, 2, 1, 3)).reshape(3, TB, NH)
    gi_t = gi3.reshape(3, T, B, NH)

    h_final = pl.pallas_call(
        _gru_kernel,
        grid=(T,),
        in_specs=[
            pl.BlockSpec(memory_space=pl.ANY),
            pl.BlockSpec((1, B, NH), lambda t: (t, 0, 0)),
            pl.BlockSpec((1, B, NH), lambda t: (t, 0, 0)),
            pl.BlockSpec((1, B, NH), lambda t: (t, 0, 0)),
            pl.BlockSpec((B, NH), lambda t: (0, 0)),
        ],
        out_specs=pl.BlockSpec((B, NH), lambda t: (0, 0)),
        out_shape=jax.ShapeDtypeStruct((B, NH), f32),
        scratch_shapes=[
            pltpu.VMEM((3, NH, NH), jnp.bfloat16),
            pltpu.VMEM((B, NH), f32),
            pltpu.SemaphoreType.DMA,
        ],
        compiler_params=pltpu.CompilerParams(
            dimension_semantics=("arbitrary",),
            vmem_limit_bytes=128 * 1024 * 1024,
        ),
        interpret=interpret,
    )(whhT, gi_t[0], gi_t[1], gi_t[2], b_n_bc)

    return h_final.reshape(B, N, H)
